# baseline (speedup 1.0000x reference)
import functools

import jax
import jax.numpy as jnp
from jax import lax
from jax.experimental import pallas as pl
from jax.experimental.pallas import tpu as pltpu
from jax.experimental.pallas import tpu_sc as plsc

IN_DIM = 1000
EMBED = 512
HID = 512
T = 200
B = 128
N = T * B
VPAD = 1008
_HH = HID // 2

_TA = 96
_TB = T - _TA
_NA = _TA * B
_NB = _TB * B


def _table_body(emb_ref, w_ref, bias_ref, out_ref):
    res = (
        lax.dot_general(
            emb_ref[...], w_ref[...], (((1,), (1,)), ((), ())),
            preferred_element_type=jnp.float32,
        )
        + bias_ref[...]
    )
    lo = lax.bitcast_convert_type(res[:, :_HH], jnp.uint32)
    hi = lax.bitcast_convert_type(res[:, _HH:], jnp.uint32)
    lo_t = (lo + 0x8000) >> 16
    hi_t = (hi + 0x8000) & jnp.uint32(0xFFFF0000)
    out_ref[...] = lax.bitcast_convert_type(hi_t | lo_t, jnp.float32)


def _make_table(emb_pad, w_ih, bias2d):
    return pl.pallas_call(
        _table_body,
        out_shape=jax.ShapeDtypeStruct((VPAD, _HH), jnp.float32),
    )(emb_pad, w_ih, bias2d)


_NC = 2
_NS = 16
_NW = _NC * _NS


@functools.cache
def _build_gather(nh):
    bpw = nh // _NW
    nchunk = 4
    ch = bpw // nchunk
    mesh = plsc.VectorSubcoreMesh(core_axis_name="c", subcore_axis_name="s")

    @functools.partial(
        pl.kernel,
        out_type=jax.ShapeDtypeStruct((nh, _HH), jnp.float32),
        mesh=mesh,
        scratch_types=[
            pltpu.VMEM((bpw,), jnp.int32),
        ]
        + [pltpu.VMEM((ch, _HH), jnp.float32) for _ in range(nchunk)]
        + [pltpu.SemaphoreType.DMA for _ in range(nchunk)]
        + [pltpu.SemaphoreType.DMA for _ in range(nchunk)],
    )
    def _gather(table_hbm, idx_hbm, out_hbm, idx_v, *bufsem):
        bufs = bufsem[:nchunk]
        gsems = bufsem[nchunk:2 * nchunk]
        wsems = bufsem[2 * nchunk:]
        wid = lax.axis_index("s") * _NC + lax.axis_index("c")
        base = wid * bpw
        pltpu.sync_copy(idx_hbm.at[pl.ds(base, bpw)], idx_v)
        copies = [
            pltpu.async_copy(
                table_hbm.at[idx_v.at[pl.ds(c * ch, ch)]], bufs[c], gsems[c]
            )
            for c in range(nchunk)
        ]
        writes = []
        for c in range(nchunk):
            copies[c].wait()
            writes.append(
                pltpu.async_copy(
                    bufs[c], out_hbm.at[pl.ds(base + c * ch, ch)], wsems[c]
                )
            )
        for w in writes:
            w.wait()

    return _gather


_UT = 8
_GA = _TA // _UT
_GB = _TB // _UT


def _fused_body(t_off, len_ref, xp_ref, whh_ref, wout_ref, b_ref, *rest):
    if t_off == 0:
        out_ref, hout_ref, h_ref = rest
    else:
        hin_ref, _outal_ref, out_ref, h_ref = rest
    i = pl.program_id(0)

    @pl.when(i == 0)
    def _():
        if t_off == 0:
            h_ref[...] = jnp.zeros_like(h_ref)
        else:
            h_ref[...] = hin_ref[...]

    h = h_ref[...]
    recs = []
    for j in range(_UT):
        t = t_off + i * _UT + j
        w = lax.bitcast_convert_type(xp_ref[pl.ds(j * B, B), :], jnp.uint32)
        x = jnp.concatenate(
            [
                lax.bitcast_convert_type(w << 16, jnp.float32),
                lax.bitcast_convert_type(w & jnp.uint32(0xFFFF0000), jnp.float32),
            ],
            axis=1,
        )
        h_new = jnp.maximum(
            x
            + lax.dot_general(
                h.astype(jnp.bfloat16), whh_ref[...], (((1,), (1,)), ((), ())),
                preferred_element_type=jnp.float32,
            ),
            0.0,
        )
        mask = len_ref[...] > t
        h = jnp.where(mask, h_new, h)
        recs.append(jnp.where(mask, h_new, 0.0).astype(jnp.bfloat16))
    h_ref[...] = h
    if t_off == 0:
        hout_ref[...] = h

    rec = jnp.concatenate(recs, axis=0)
    xt = rec.T
    logits = (
        jnp.dot(wout_ref[...], xt, preferred_element_type=jnp.float32)
        + b_ref[...]
    )
    m = jnp.max(logits, axis=0, keepdims=True)
    e = jnp.exp(logits - m)
    s = jnp.sum(e, axis=0, keepdims=True)
    res = logits - m - jnp.log(s)
    for j in range(_UT):
        out_ref[j] = res[:, j * B:(j + 1) * B]


_W_SPECS = [
    pl.BlockSpec((B, 1), lambda i: (0, 0)),
    pl.BlockSpec((_UT * B, _HH), lambda i: (i, 0)),
    pl.BlockSpec((HID, HID), lambda i: (0, 0)),
    pl.BlockSpec((IN_DIM, HID), lambda i: (0, 0)),
    pl.BlockSpec((IN_DIM, 1), lambda i: (0, 0)),
]


def _run_fused_a(lengths2d, xp1, w_hh_bf16, w_out_bf16, b_out2d):
    return pl.pallas_call(
        functools.partial(_fused_body, 0),
        grid=(_GA,),
        in_specs=_W_SPECS,
        out_specs=[
            pl.BlockSpec((_UT, IN_DIM, B), lambda i: (i, 0, 0)),
            pl.BlockSpec((B, HID), lambda i: (0, 0)),
        ],
        out_shape=[
            jax.ShapeDtypeStruct((T, IN_DIM, B), jnp.float32),
            jax.ShapeDtypeStruct((B, HID), jnp.float32),
        ],
        scratch_shapes=[pltpu.VMEM((B, HID), jnp.float32)],
    )(lengths2d, xp1, w_hh_bf16, w_out_bf16, b_out2d)


def _run_fused_b(lengths2d, xp2, w_hh_bf16, w_out_bf16, b_out2d, h_mid, out_alias):
    return pl.pallas_call(
        functools.partial(_fused_body, _TA),
        grid=(_GB,),
        in_specs=_W_SPECS
        + [
            pl.BlockSpec((B, HID), lambda i: (0, 0)),
            pl.BlockSpec(memory_space=pl.ANY),
        ],
        out_specs=pl.BlockSpec((_UT, IN_DIM, B), lambda i: (i + _GA, 0, 0)),
        out_shape=jax.ShapeDtypeStruct((T, IN_DIM, B), jnp.float32),
        input_output_aliases={6: 0},
        scratch_shapes=[pltpu.VMEM((B, HID), jnp.float32)],
    )(lengths2d, xp2, w_hh_bf16, w_out_bf16, b_out2d, h_mid, out_alias)


def kernel(inputs, lengths, embed, W_ih, W_hh, b_ih, b_hh, W_out, b_out):
    emb_pad = jnp.pad(embed, ((0, VPAD - (IN_DIM + 1)), (0, 0)))
    bias2d = (b_ih + b_hh).reshape(1, HID)
    table = _make_table(emb_pad, W_ih, bias2d)

    idx = inputs.reshape(N).astype(jnp.int32)
    xp1 = _build_gather(_NA)(table, idx[:_NA])
    xp2 = _build_gather(_NB)(table, idx[_NA:])

    lengths2d = lengths.reshape(B, 1).astype(jnp.int32)
    w_hh_bf16 = W_hh.astype(jnp.bfloat16)
    w_out_bf16 = W_out.astype(jnp.bfloat16)
    b_out2d = b_out.reshape(IN_DIM, 1)

    out_a, h_mid = _run_fused_a(lengths2d, xp1, w_hh_bf16, w_out_bf16, b_out2d)
    out_tvb = _run_fused_b(
        lengths2d, xp2, w_hh_bf16, w_out_bf16, b_out2d, h_mid, out_a
    )
    return out_tvb.transpose(0, 2, 1)


# speedup vs baseline: 1.0432x; 1.0432x over previous
import functools

import jax
import jax.numpy as jnp
from jax import lax
from jax.experimental import pallas as pl
from jax.experimental.pallas import tpu as pltpu
from jax.experimental.pallas import tpu_sc as plsc

IN_DIM = 1000
EMBED = 512
HID = 512
T = 200
B = 128
N = T * B
VPAD = 1008
_HH = HID // 2

_TA = 96
_TB = T - _TA
_NA = _TA * B
_NB = _TB * B


def _table_body(emb_ref, w_ref, bias_ref, out_ref):
    res = (
        lax.dot_general(
            emb_ref[...], w_ref[...], (((1,), (1,)), ((), ())),
            preferred_element_type=jnp.float32,
        )
        + bias_ref[...]
    )
    lo = lax.bitcast_convert_type(res[:, :_HH], jnp.uint32)
    hi = lax.bitcast_convert_type(res[:, _HH:], jnp.uint32)
    lo_t = (lo + 0x8000) >> 16
    hi_t = (hi + 0x8000) & jnp.uint32(0xFFFF0000)
    out_ref[...] = lax.bitcast_convert_type(hi_t | lo_t, jnp.float32)


def _make_table(emb_pad, w_ih, bias2d):
    return pl.pallas_call(
        _table_body,
        out_shape=jax.ShapeDtypeStruct((VPAD, _HH), jnp.float32),
    )(emb_pad, w_ih, bias2d)


_NC = 2
_NS = 16
_NW = _NC * _NS


@functools.cache
def _build_gather(nh):
    bpw = nh // _NW
    nchunk = 4
    ch = bpw // nchunk
    mesh = plsc.VectorSubcoreMesh(core_axis_name="c", subcore_axis_name="s")

    @functools.partial(
        pl.kernel,
        out_type=jax.ShapeDtypeStruct((nh, _HH), jnp.float32),
        mesh=mesh,
        scratch_types=[
            pltpu.VMEM((bpw,), jnp.int32),
        ]
        + [pltpu.VMEM((ch, _HH), jnp.float32) for _ in range(nchunk)]
        + [pltpu.SemaphoreType.DMA for _ in range(nchunk)]
        + [pltpu.SemaphoreType.DMA for _ in range(nchunk)],
    )
    def _gather(table_hbm, idx_hbm, out_hbm, idx_v, *bufsem):
        bufs = bufsem[:nchunk]
        gsems = bufsem[nchunk:2 * nchunk]
        wsems = bufsem[2 * nchunk:]
        wid = lax.axis_index("s") * _NC + lax.axis_index("c")
        base = wid * bpw
        pltpu.sync_copy(idx_hbm.at[pl.ds(base, bpw)], idx_v)
        copies = [
            pltpu.async_copy(
                table_hbm.at[idx_v.at[pl.ds(c * ch, ch)]], bufs[c], gsems[c]
            )
            for c in range(nchunk)
        ]
        writes = []
        for c in range(nchunk):
            copies[c].wait()
            writes.append(
                pltpu.async_copy(
                    bufs[c], out_hbm.at[pl.ds(base + c * ch, ch)], wsems[c]
                )
            )
        for w in writes:
            w.wait()

    return _gather


_UT = 8
_GA = _TA // _UT
_GB = _TB // _UT


def _fused_body(t_off, len_ref, xp_ref, whh_ref, wout_ref, b_ref, *rest):
    if t_off == 0:
        out_ref, hout_ref, h_ref = rest
    else:
        hin_ref, _outal_ref, out_ref, h_ref = rest
    i = pl.program_id(0)

    @pl.when(i == 0)
    def _():
        if t_off == 0:
            h_ref[...] = jnp.zeros_like(h_ref)
        else:
            h_ref[...] = hin_ref[...]

    h = h_ref[...]
    recs = []
    for j in range(_UT):
        t = t_off + i * _UT + j
        w = lax.bitcast_convert_type(xp_ref[pl.ds(j * B, B), :], jnp.uint32)
        x = jnp.concatenate(
            [
                lax.bitcast_convert_type(w << 16, jnp.float32),
                lax.bitcast_convert_type(w & jnp.uint32(0xFFFF0000), jnp.float32),
            ],
            axis=1,
        )
        h_new = jnp.maximum(
            x
            + jnp.dot(
                h.astype(jnp.bfloat16), whh_ref[...],
                preferred_element_type=jnp.float32,
            ),
            0.0,
        )
        mask = len_ref[...] > t
        h = jnp.where(mask, h_new, h)
        recs.append(jnp.where(mask, h_new, 0.0).astype(jnp.bfloat16))
    h_ref[...] = h
    if t_off == 0:
        hout_ref[...] = h

    rec = jnp.concatenate(recs, axis=0)
    xt = rec.T
    logits = (
        jnp.dot(wout_ref[...], xt, preferred_element_type=jnp.float32)
        + b_ref[...]
    )
    m = jnp.max(logits, axis=0, keepdims=True)
    e = jnp.exp(logits - m)
    s = jnp.sum(e, axis=0, keepdims=True)
    res = logits - m - jnp.log(s)
    for j in range(_UT):
        out_ref[j] = res[:, j * B:(j + 1) * B]


_W_SPECS = [
    pl.BlockSpec((B, 1), lambda i: (0, 0)),
    pl.BlockSpec((_UT * B, _HH), lambda i: (i, 0)),
    pl.BlockSpec((HID, HID), lambda i: (0, 0)),
    pl.BlockSpec((IN_DIM, HID), lambda i: (0, 0)),
    pl.BlockSpec((IN_DIM, 1), lambda i: (0, 0)),
]


def _run_fused_a(lengths2d, xp1, w_hh_bf16, w_out_bf16, b_out2d):
    return pl.pallas_call(
        functools.partial(_fused_body, 0),
        grid=(_GA,),
        in_specs=_W_SPECS,
        out_specs=[
            pl.BlockSpec((_UT, IN_DIM, B), lambda i: (i, 0, 0)),
            pl.BlockSpec((B, HID), lambda i: (0, 0)),
        ],
        out_shape=[
            jax.ShapeDtypeStruct((T, IN_DIM, B), jnp.float32),
            jax.ShapeDtypeStruct((B, HID), jnp.float32),
        ],
        scratch_shapes=[pltpu.VMEM((B, HID), jnp.float32)],
    )(lengths2d, xp1, w_hh_bf16, w_out_bf16, b_out2d)


def _run_fused_b(lengths2d, xp2, w_hh_bf16, w_out_bf16, b_out2d, h_mid, out_alias):
    return pl.pallas_call(
        functools.partial(_fused_body, _TA),
        grid=(_GB,),
        in_specs=_W_SPECS
        + [
            pl.BlockSpec((B, HID), lambda i: (0, 0)),
            pl.BlockSpec(memory_space=pl.ANY),
        ],
        out_specs=pl.BlockSpec((_UT, IN_DIM, B), lambda i: (i + _GA, 0, 0)),
        out_shape=jax.ShapeDtypeStruct((T, IN_DIM, B), jnp.float32),
        input_output_aliases={6: 0},
        scratch_shapes=[pltpu.VMEM((B, HID), jnp.float32)],
    )(lengths2d, xp2, w_hh_bf16, w_out_bf16, b_out2d, h_mid, out_alias)


def kernel(inputs, lengths, embed, W_ih, W_hh, b_ih, b_hh, W_out, b_out):
    emb_pad = jnp.pad(embed, ((0, VPAD - (IN_DIM + 1)), (0, 0)))
    bias2d = (b_ih + b_hh).reshape(1, HID)
    table = _make_table(emb_pad, W_ih, bias2d)

    idx = inputs.reshape(N).astype(jnp.int32)
    xp1 = _build_gather(_NA)(table, idx[:_NA])
    xp2 = _build_gather(_NB)(table, idx[_NA:])

    lengths2d = lengths.reshape(B, 1).astype(jnp.int32)
    w_hh_bf16 = W_hh.T.astype(jnp.bfloat16)
    w_out_bf16 = W_out.astype(jnp.bfloat16)
    b_out2d = b_out.reshape(IN_DIM, 1)

    out_a, h_mid = _run_fused_a(lengths2d, xp1, w_hh_bf16, w_out_bf16, b_out2d)
    out_tvb = _run_fused_b(
        lengths2d, xp2, w_hh_bf16, w_out_bf16, b_out2d, h_mid, out_a
    )
    return out_tvb.transpose(0, 2, 1)


# speedup vs baseline: 1.0968x; 1.0513x over previous
import functools

import jax
import jax.numpy as jnp
from jax import lax
from jax.experimental import pallas as pl
from jax.experimental.pallas import tpu as pltpu
from jax.experimental.pallas import tpu_sc as plsc

IN_DIM = 1000
EMBED = 512
HID = 512
T = 200
B = 128
N = T * B
VPAD = 1008
_HH = HID // 2

_TA = 96
_TB = T - _TA
_NA = _TA * B
_NB = _TB * B


def _table_body(emb_ref, w_ref, bih_ref, bhh_ref, out_ref):
    res = (
        lax.dot_general(
            emb_ref[...], w_ref[...], (((1,), (1,)), ((), ())),
            preferred_element_type=jnp.float32,
        )
        + bih_ref[...]
        + bhh_ref[...]
    )
    lo = lax.bitcast_convert_type(res[:, :_HH], jnp.uint32)
    hi = lax.bitcast_convert_type(res[:, _HH:], jnp.uint32)
    lo_t = (lo + 0x8000) >> 16
    hi_t = (hi + 0x8000) & jnp.uint32(0xFFFF0000)
    out_ref[...] = lax.bitcast_convert_type(hi_t | lo_t, jnp.float32)


def _make_table(embed, w_ih, bih2d, bhh2d):
    return pl.pallas_call(
        _table_body,
        grid=(1,),
        in_specs=[
            pl.BlockSpec((VPAD, EMBED), lambda i: (0, 0)),
            pl.BlockSpec((HID, EMBED), lambda i: (0, 0)),
            pl.BlockSpec((1, HID), lambda i: (0, 0)),
            pl.BlockSpec((1, HID), lambda i: (0, 0)),
        ],
        out_specs=pl.BlockSpec((VPAD, _HH), lambda i: (0, 0)),
        out_shape=jax.ShapeDtypeStruct((VPAD, _HH), jnp.float32),
    )(embed, w_ih, bih2d, bhh2d)


_NC = 2
_NS = 16
_NW = _NC * _NS


@functools.cache
def _build_gather(nh, off):
    bpw = nh // _NW
    nchunk = 8 if (bpw // 8) % 8 == 0 else 4
    ch = bpw // nchunk
    mesh = plsc.VectorSubcoreMesh(core_axis_name="c", subcore_axis_name="s")

    @functools.partial(
        pl.kernel,
        out_type=jax.ShapeDtypeStruct((nh, _HH), jnp.float32),
        mesh=mesh,
        scratch_types=[
            pltpu.VMEM((bpw,), jnp.int32),
        ]
        + [pltpu.VMEM((ch, _HH), jnp.float32) for _ in range(nchunk)]
        + [pltpu.SemaphoreType.DMA for _ in range(nchunk)]
        + [pltpu.SemaphoreType.DMA for _ in range(nchunk)],
    )
    def _gather(table_hbm, idx_hbm, out_hbm, idx_v, *bufsem):
        bufs = bufsem[:nchunk]
        gsems = bufsem[nchunk:2 * nchunk]
        wsems = bufsem[2 * nchunk:]
        wid = lax.axis_index("s") * _NC + lax.axis_index("c")
        base = wid * bpw
        pltpu.sync_copy(idx_hbm.at[pl.ds(off + base, bpw)], idx_v)
        copies = [
            pltpu.async_copy(
                table_hbm.at[idx_v.at[pl.ds(c * ch, ch)]], bufs[c], gsems[c]
            )
            for c in range(nchunk)
        ]
        writes = []
        for c in range(nchunk):
            copies[c].wait()
            writes.append(
                pltpu.async_copy(
                    bufs[c], out_hbm.at[pl.ds(base + c * ch, ch)], wsems[c]
                )
            )
        for w in writes:
            w.wait()

    return _gather


_UT = 8
_GA = _TA // _UT
_GB = _TB // _UT


def _fused_body(t_off, len_ref, xp_ref, whh_ref, wout_ref, b_ref, *rest):
    if t_off == 0:
        out_ref, hout_ref, h_ref = rest
    else:
        hin_ref, _outal_ref, out_ref, h_ref = rest
    i = pl.program_id(0)

    @pl.when(i == 0)
    def _():
        if t_off == 0:
            h_ref[...] = jnp.zeros_like(h_ref)
        else:
            h_ref[...] = hin_ref[...]

    h = h_ref[...]
    recs = []
    for j in range(_UT):
        t = t_off + i * _UT + j
        w = lax.bitcast_convert_type(xp_ref[pl.ds(j * B, B), :], jnp.uint32)
        x = jnp.concatenate(
            [
                lax.bitcast_convert_type(w << 16, jnp.float32),
                lax.bitcast_convert_type(w & jnp.uint32(0xFFFF0000), jnp.float32),
            ],
            axis=1,
        )
        h_new = jnp.maximum(
            x
            + jnp.dot(
                h.astype(jnp.bfloat16), whh_ref[...],
                preferred_element_type=jnp.float32,
            ),
            0.0,
        )
        mask = len_ref[...] > t
        h = jnp.where(mask, h_new, h)
        recs.append(jnp.where(mask, h_new, 0.0).astype(jnp.bfloat16))
    h_ref[...] = h
    if t_off == 0:
        hout_ref[...] = h

    rec = jnp.concatenate(recs, axis=0)
    xt = rec.T
    logits = (
        jnp.dot(wout_ref[...], xt, preferred_element_type=jnp.float32)
        + b_ref[...]
    )
    m = jnp.max(logits, axis=0, keepdims=True)
    e = jnp.exp(logits - m)
    s = jnp.sum(e, axis=0, keepdims=True)
    res = logits - m - jnp.log(s)
    for j in range(_UT):
        out_ref[j] = res[:, j * B:(j + 1) * B]


_W_SPECS = [
    pl.BlockSpec((B, 1), lambda i: (0, 0)),
    pl.BlockSpec((_UT * B, _HH), lambda i: (i, 0)),
    pl.BlockSpec((HID, HID), lambda i: (0, 0)),
    pl.BlockSpec((IN_DIM, HID), lambda i: (0, 0)),
    pl.BlockSpec((IN_DIM, 1), lambda i: (0, 0)),
]


def _run_fused_a(lengths2d, xp1, w_hh_bf16, w_out_bf16, b_out2d):
    return pl.pallas_call(
        functools.partial(_fused_body, 0),
        grid=(_GA,),
        in_specs=_W_SPECS,
        out_specs=[
            pl.BlockSpec((_UT, IN_DIM, B), lambda i: (i, 0, 0)),
            pl.BlockSpec((B, HID), lambda i: (0, 0)),
        ],
        out_shape=[
            jax.ShapeDtypeStruct((T, IN_DIM, B), jnp.float32),
            jax.ShapeDtypeStruct((B, HID), jnp.float32),
        ],
        scratch_shapes=[pltpu.VMEM((B, HID), jnp.float32)],
    )(lengths2d, xp1, w_hh_bf16, w_out_bf16, b_out2d)


def _run_fused_b(lengths2d, xp2, w_hh_bf16, w_out_bf16, b_out2d, h_mid, out_alias):
    return pl.pallas_call(
        functools.partial(_fused_body, _TA),
        grid=(_GB,),
        in_specs=_W_SPECS
        + [
            pl.BlockSpec((B, HID), lambda i: (0, 0)),
            pl.BlockSpec(memory_space=pl.ANY),
        ],
        out_specs=pl.BlockSpec((_UT, IN_DIM, B), lambda i: (i + _GA, 0, 0)),
        out_shape=jax.ShapeDtypeStruct((T, IN_DIM, B), jnp.float32),
        input_output_aliases={6: 0},
        scratch_shapes=[pltpu.VMEM((B, HID), jnp.float32)],
    )(lengths2d, xp2, w_hh_bf16, w_out_bf16, b_out2d, h_mid, out_alias)


def kernel(inputs, lengths, embed, W_ih, W_hh, b_ih, b_hh, W_out, b_out):
    table = _make_table(
        embed, W_ih, b_ih.reshape(1, HID), b_hh.reshape(1, HID)
    )

    idx = inputs.reshape(N).astype(jnp.int32)
    xp1 = _build_gather(_NA, 0)(table, idx)
    xp2 = _build_gather(_NB, _NA)(table, idx)

    lengths2d = lengths.reshape(B, 1).astype(jnp.int32)
    w_hh_bf16 = W_hh.T.astype(jnp.bfloat16)
    w_out_bf16 = W_out.astype(jnp.bfloat16)
    b_out2d = b_out.reshape(IN_DIM, 1)

    out_a, h_mid = _run_fused_a(lengths2d, xp1, w_hh_bf16, w_out_bf16, b_out2d)
    out_tvb = _run_fused_b(
        lengths2d, xp2, w_hh_bf16, w_out_bf16, b_out2d, h_mid, out_a
    )
    return out_tvb.transpose(0, 2, 1)
